# baseline (speedup 1.0000x reference)
import functools

import jax
import jax.numpy as jnp
from jax import lax
from jax.experimental import pallas as pl
from jax.experimental.pallas import tpu as pltpu
from jax.experimental.pallas import tpu_sc as plsc

E = 8
TOP_K = 2
D = 768
FFN = 3072
T = 2048
CAP = 320
CPAD = 320
ROWS = E * CPAD
NW = 32
TPW = T // NW
BUF_ROWS = ROWS + NW
FB = 768
NF = FFN // FB


def _router_body(x_ref, wg_ref, d0s_ref, d1s_ref,
                 wk0_ref, wk1_ref, aux_ref, cnt_ref, r_ref):
    logits = lax.dot_general(wg_ref[...], x_ref[...],
                             (((1,), (1,)), ((), ())),
                             preferred_element_type=jnp.float32)
    e_iota = lax.broadcasted_iota(jnp.int32, (E, T), 0).astype(jnp.float32)

    l0 = jnp.max(logits, axis=0, keepdims=True)
    is0 = logits == l0
    e0 = jnp.min(jnp.where(is0, e_iota, float(E)), axis=0, keepdims=True)
    logits1 = jnp.where(e_iota == e0, -jnp.inf, logits)
    l1 = jnp.max(logits1, axis=0, keepdims=True)
    e1 = jnp.min(jnp.where(logits1 == l1, e_iota, float(E)), axis=0,
                 keepdims=True)

    w0 = 1.0 / (1.0 + jnp.exp(l1 - l0))
    w1 = 1.0 - w0

    p = jnp.exp(logits - l0)
    p = p / jnp.sum(p, axis=0, keepdims=True)
    probs_mean = jnp.sum(p, axis=1, keepdims=True) / float(T)

    oh0 = (e_iota == e0).astype(jnp.float32)
    oh1 = (e_iota == e1).astype(jnp.float32)
    cnt = oh0 + oh1
    cnt_ref[...] = cnt
    tpe = jnp.sum(cnt, axis=1, keepdims=True)
    aux_ref[...] = 0.01 * float(E) * jnp.sum(tpe / float(T) * probs_mean,
                                             axis=0, keepdims=True)

    ri = lax.broadcasted_iota(jnp.int32, (128, 128), 0)
    ci = lax.broadcasted_iota(jnp.int32, (128, 128), 1)
    upper = (ri < ci).astype(jnp.float32)

    def chunk(c, carry):
        blk = cnt_ref[:, pl.ds(c * 128, 128)]
        r_ref[:, pl.ds(c * 128, 128)] = carry + lax.dot_general(
            blk, upper, (((1,), (0,)), ((), ())),
            preferred_element_type=jnp.float32)
        return carry + jnp.sum(blk, axis=1, keepdims=True)

    lax.fori_loop(0, T // 128, chunk, jnp.zeros((E, 1), jnp.float32))

    ranks = r_ref[...]
    r0 = jnp.sum(ranks * oh0, axis=0, keepdims=True)
    r1 = jnp.sum(ranks * oh1, axis=0, keepdims=True)
    keep0 = r0 < float(CAP)
    keep1 = r1 < float(CAP)

    slot0 = (e0 * float(CPAD) + r0).astype(jnp.int32)
    slot1 = (e1 * float(CPAD) + r1).astype(jnp.int32)
    t_iota = lax.broadcasted_iota(jnp.int32, (1, T), 1)
    dump = ROWS + t_iota // TPW
    d0s_ref[...] = jnp.where(keep0, slot0, dump)
    d1s_ref[...] = jnp.where(keep1, slot1, dump)

    wk0_ref[...] = jnp.where(keep0, w0, 0.0)
    wk1_ref[...] = jnp.where(keep1, w1, 0.0)


def _run_router(x_flat, Wg):
    i32, f32 = jnp.int32, jnp.float32
    outs = pl.pallas_call(
        _router_body,
        out_shape=(
            jax.ShapeDtypeStruct((1, T), i32),
            jax.ShapeDtypeStruct((1, T), i32),
            jax.ShapeDtypeStruct((1, T), f32),
            jax.ShapeDtypeStruct((1, T), f32),
            jax.ShapeDtypeStruct((1, 1), f32),
        ),
        scratch_shapes=[pltpu.VMEM((E, T), f32), pltpu.VMEM((E, T), f32)],
    )(x_flat, Wg)
    return outs


@functools.cache
def _make_dispatch():
    mesh = plsc.VectorSubcoreMesh(core_axis_name="c", subcore_axis_name="s")

    @functools.partial(
        pl.kernel,
        out_type=jax.ShapeDtypeStruct((BUF_ROWS, D), jnp.float32),
        mesh=mesh,
        scratch_types=[
            pltpu.VMEM((TPW,), jnp.int32),
            pltpu.VMEM((TPW,), jnp.int32),
            pltpu.VMEM((TPW, D), jnp.float32),
            pltpu.SemaphoreType.DMA,
        ],
    )
    def _dispatch(x_hbm, d0_hbm, d1_hbm, buf_hbm, i0_v, i1_v, rows_v, sem):
        wid = lax.axis_index("s") * 2 + lax.axis_index("c")
        base = wid * TPW
        pltpu.sync_copy(d0_hbm.at[pl.ds(base, TPW)], i0_v)
        pltpu.sync_copy(d1_hbm.at[pl.ds(base, TPW)], i1_v)
        pltpu.sync_copy(x_hbm.at[pl.ds(base, TPW)], rows_v)
        c0 = pltpu.async_copy(rows_v, buf_hbm.at[i0_v], sem)
        c1 = pltpu.async_copy(rows_v, buf_hbm.at[i1_v], sem)
        c0.wait()
        c1.wait()

    return _dispatch


def _ffn_body(buf_ref, w1_ref, w3_ref, w2_ref, ds0_ref, ds1_ref,
              wk0_ref, wk1_ref, out_ref, acc_ref):
    e = pl.program_id(0)
    f = pl.program_id(1)
    xb = buf_ref[...]
    h1 = lax.dot_general(xb, w1_ref[0], (((1,), (1,)), ((), ())),
                         preferred_element_type=jnp.float32)
    h3 = lax.dot_general(xb, w3_ref[0], (((1,), (1,)), ((), ())),
                         preferred_element_type=jnp.float32)
    g = h1 * jax.nn.sigmoid(h1) * h3
    part = lax.dot_general(g, w2_ref[0], (((1,), (1,)), ((), ())),
                           preferred_element_type=jnp.float32)

    @pl.when(f == 0)
    def _():
        acc_ref[...] = part

    @pl.when(f != 0)
    def _():
        acc_ref[...] += part

    @pl.when(f == NF - 1)
    def _():
        slot = (lax.broadcasted_iota(jnp.int32, (1, CPAD), 1) + e * CPAD)
        p0 = ds0_ref[...] == slot
        p1 = ds1_ref[...] == slot
        w = (lax.dot_general(p0.astype(jnp.float32), wk0_ref[...],
                             (((0,), (0,)), ((), ())),
                             preferred_element_type=jnp.float32)
             + lax.dot_general(p1.astype(jnp.float32), wk1_ref[...],
                               (((0,), (0,)), ((), ())),
                               preferred_element_type=jnp.float32))
        yw = jnp.where(w != 0.0, acc_ref[...] * w, 0.0)
        yw16 = yw.astype(jnp.bfloat16)
        p16 = (p0 | p1).astype(jnp.bfloat16)
        contrib = lax.dot_general(p16, yw16, (((1,), (0,)), ((), ())),
                                  preferred_element_type=jnp.float32)

        @pl.when(e == 0)
        def _():
            out_ref[...] = contrib

        @pl.when(e != 0)
        def _():
            out_ref[...] += contrib


def _run_ffn_combine(buf, W1, W3, W2, ds0, ds1, wk0, wk1):
    out = pl.pallas_call(
        _ffn_body,
        grid=(E, NF),
        in_specs=[
            pl.BlockSpec((CPAD, D), lambda e, f: (e, 0)),
            pl.BlockSpec((1, FB, D), lambda e, f: (e, f, 0)),
            pl.BlockSpec((1, FB, D), lambda e, f: (e, f, 0)),
            pl.BlockSpec((1, D, FB), lambda e, f: (e, 0, f)),
            pl.BlockSpec((T, 1), lambda e, f: (0, 0)),
            pl.BlockSpec((T, 1), lambda e, f: (0, 0)),
            pl.BlockSpec((T, 1), lambda e, f: (0, 0)),
            pl.BlockSpec((T, 1), lambda e, f: (0, 0)),
        ],
        out_specs=pl.BlockSpec((T, D), lambda e, f: (0, 0)),
        out_shape=jax.ShapeDtypeStruct((T, D), jnp.float32),
        scratch_shapes=[pltpu.VMEM((CPAD, D), jnp.float32)],
    )(buf, W1, W3, W2, ds0, ds1, wk0, wk1)
    return out


def kernel(x, Wg, W1, W3, W2):
    bsz, seq, _ = x.shape
    x_flat = x.reshape(T, D)
    d0s, d1s, wk0, wk1, aux = _run_router(x_flat, Wg)
    buf = _make_dispatch()(x_flat, d0s.reshape(T), d1s.reshape(T))
    out = _run_ffn_combine(buf, W1, W3, W2,
                           d0s.reshape(T, 1), d1s.reshape(T, 1),
                           wk0.reshape(T, 1), wk1.reshape(T, 1))
    return out.reshape(bsz, seq, D), aux[0, 0]


# speedup vs baseline: 1.1000x; 1.1000x over previous
import functools

import jax
import jax.numpy as jnp
from jax import lax
from jax.experimental import pallas as pl
from jax.experimental.pallas import tpu as pltpu
from jax.experimental.pallas import tpu_sc as plsc

E = 8
TOP_K = 2
D = 768
FFN = 3072
T = 2048
CAP = 320
CPAD = 320
ROWS = E * CPAD
NW = 32
TPW = T // NW
BUF_ROWS = ROWS + NW
FB = 768
NF = FFN // FB


def _router_body(x_ref, wg_ref, d0s_ref, d1s_ref,
                 wslot_ref, aux_ref, cnt_ref, r_ref):
    logits = lax.dot_general(wg_ref[...], x_ref[...],
                             (((1,), (1,)), ((), ())),
                             preferred_element_type=jnp.float32)
    e_iota = lax.broadcasted_iota(jnp.int32, (E, T), 0).astype(jnp.float32)

    l0 = jnp.max(logits, axis=0, keepdims=True)
    is0 = logits == l0
    e0 = jnp.min(jnp.where(is0, e_iota, float(E)), axis=0, keepdims=True)
    logits1 = jnp.where(e_iota == e0, -jnp.inf, logits)
    l1 = jnp.max(logits1, axis=0, keepdims=True)
    e1 = jnp.min(jnp.where(logits1 == l1, e_iota, float(E)), axis=0,
                 keepdims=True)

    w0 = 1.0 / (1.0 + jnp.exp(l1 - l0))
    w1 = 1.0 - w0

    p = jnp.exp(logits - l0)
    p = p / jnp.sum(p, axis=0, keepdims=True)
    probs_mean = jnp.sum(p, axis=1, keepdims=True) / float(T)

    oh0 = (e_iota == e0).astype(jnp.float32)
    oh1 = (e_iota == e1).astype(jnp.float32)
    cnt = oh0 + oh1
    cnt_ref[...] = cnt
    tpe = jnp.sum(cnt, axis=1, keepdims=True)
    aux_ref[...] = 0.01 * float(E) * jnp.sum(tpe / float(T) * probs_mean,
                                             axis=0, keepdims=True)

    ri = lax.broadcasted_iota(jnp.int32, (128, 128), 0)
    ci = lax.broadcasted_iota(jnp.int32, (128, 128), 1)
    upper = (ri < ci).astype(jnp.float32)

    def chunk(c, carry):
        blk = cnt_ref[:, pl.ds(c * 128, 128)]
        r_ref[:, pl.ds(c * 128, 128)] = carry + lax.dot_general(
            blk, upper, (((1,), (0,)), ((), ())),
            preferred_element_type=jnp.float32)
        return carry + jnp.sum(blk, axis=1, keepdims=True)

    lax.fori_loop(0, T // 128, chunk, jnp.zeros((E, 1), jnp.float32))

    ranks = r_ref[...]
    r0 = jnp.sum(ranks * oh0, axis=0, keepdims=True)
    r1 = jnp.sum(ranks * oh1, axis=0, keepdims=True)
    keep0 = r0 < float(CAP)
    keep1 = r1 < float(CAP)

    slot0 = (e0 * float(CPAD) + r0).astype(jnp.int32)
    slot1 = (e1 * float(CPAD) + r1).astype(jnp.int32)
    t_iota = lax.broadcasted_iota(jnp.int32, (1, T), 1)
    dump = ROWS + t_iota // TPW
    d0s_ref[...] = jnp.where(keep0, slot0, dump)
    d1s_ref[...] = jnp.where(keep1, slot1, dump)

    wk0 = jnp.where(keep0, w0, 0.0)
    wk1 = jnp.where(keep1, w1, 0.0)
    s_iota = lax.broadcasted_iota(jnp.int32, (CPAD, 1), 0)
    for e in range(E):
        sid = s_iota + e * CPAD
        c0 = sid == d0s_ref[...]
        c1 = sid == d1s_ref[...]
        w = jnp.where(c0, wk0, 0.0) + jnp.where(c1, wk1, 0.0)
        wslot_ref[pl.ds(e * CPAD, CPAD), :] = jnp.sum(w, axis=1, keepdims=True)


def _run_router(x_flat, Wg):
    i32, f32 = jnp.int32, jnp.float32
    outs = pl.pallas_call(
        _router_body,
        out_shape=(
            jax.ShapeDtypeStruct((1, T), i32),
            jax.ShapeDtypeStruct((1, T), i32),
            jax.ShapeDtypeStruct((ROWS, 1), f32),
            jax.ShapeDtypeStruct((1, 1), f32),
        ),
        scratch_shapes=[pltpu.VMEM((E, T), f32), pltpu.VMEM((E, T), f32)],
    )(x_flat, Wg)
    return outs


@functools.cache
def _make_dispatch():
    mesh = plsc.VectorSubcoreMesh(core_axis_name="c", subcore_axis_name="s")

    @functools.partial(
        pl.kernel,
        out_type=jax.ShapeDtypeStruct((BUF_ROWS, D), jnp.float32),
        mesh=mesh,
        scratch_types=[
            pltpu.VMEM((TPW,), jnp.int32),
            pltpu.VMEM((TPW,), jnp.int32),
            pltpu.VMEM((TPW, D), jnp.float32),
            pltpu.SemaphoreType.DMA,
        ],
    )
    def _dispatch(x_hbm, d0_hbm, d1_hbm, buf_hbm, i0_v, i1_v, rows_v, sem):
        wid = lax.axis_index("s") * 2 + lax.axis_index("c")
        base = wid * TPW
        pltpu.sync_copy(d0_hbm.at[pl.ds(base, TPW)], i0_v)
        pltpu.sync_copy(d1_hbm.at[pl.ds(base, TPW)], i1_v)
        pltpu.sync_copy(x_hbm.at[pl.ds(base, TPW)], rows_v)
        pltpu.async_copy(rows_v, buf_hbm.at[i0_v], sem).wait()
        pltpu.async_copy(rows_v, buf_hbm.at[i1_v], sem).wait()

    return _dispatch


def _ffn_body(buf_ref, w1_ref, w3_ref, w2_ref, wcol_ref, ds0_ref, ds1_ref,
              out_ref, acc_ref):
    e = pl.program_id(0)
    f = pl.program_id(1)
    xb = buf_ref[...]
    h1 = lax.dot_general(xb, w1_ref[0], (((1,), (1,)), ((), ())),
                         preferred_element_type=jnp.float32)
    h3 = lax.dot_general(xb, w3_ref[0], (((1,), (1,)), ((), ())),
                         preferred_element_type=jnp.float32)
    g = h1 * jax.nn.sigmoid(h1) * h3
    part = lax.dot_general(g, w2_ref[0], (((1,), (1,)), ((), ())),
                           preferred_element_type=jnp.float32)

    @pl.when(f == 0)
    def _():
        acc_ref[...] = part

    @pl.when(f != 0)
    def _():
        acc_ref[...] += part

    @pl.when(f == NF - 1)
    def _():
        w = wcol_ref[...]
        yw = jnp.where(w != 0.0, acc_ref[...] * w, 0.0)
        yw16 = yw.astype(jnp.bfloat16)
        slot = (lax.broadcasted_iota(jnp.int32, (1, CPAD), 1) + e * CPAD)
        pmat = ((ds0_ref[...] == slot) | (ds1_ref[...] == slot))
        p16 = pmat.astype(jnp.bfloat16)
        contrib = lax.dot_general(p16, yw16, (((1,), (0,)), ((), ())),
                                  preferred_element_type=jnp.float32)

        @pl.when(e == 0)
        def _():
            out_ref[...] = contrib

        @pl.when(e != 0)
        def _():
            out_ref[...] += contrib


def _run_ffn_combine(buf, W1, W3, W2, wslot, ds0, ds1):
    out = pl.pallas_call(
        _ffn_body,
        grid=(E, NF),
        in_specs=[
            pl.BlockSpec((CPAD, D), lambda e, f: (e, 0)),
            pl.BlockSpec((1, FB, D), lambda e, f: (e, f, 0)),
            pl.BlockSpec((1, FB, D), lambda e, f: (e, f, 0)),
            pl.BlockSpec((1, D, FB), lambda e, f: (e, 0, f)),
            pl.BlockSpec((CPAD, 1), lambda e, f: (e, 0)),
            pl.BlockSpec((T, 1), lambda e, f: (0, 0)),
            pl.BlockSpec((T, 1), lambda e, f: (0, 0)),
        ],
        out_specs=pl.BlockSpec((T, D), lambda e, f: (0, 0)),
        out_shape=jax.ShapeDtypeStruct((T, D), jnp.float32),
        scratch_shapes=[pltpu.VMEM((CPAD, D), jnp.float32)],
    )(buf, W1, W3, W2, wslot, ds0, ds1)
    return out


def kernel(x, Wg, W1, W3, W2):
    bsz, seq, _ = x.shape
    x_flat = x.reshape(T, D)
    d0s, d1s, wslot, aux = _run_router(x_flat, Wg)
    buf = _make_dispatch()(x_flat, d0s.reshape(T), d1s.reshape(T))
    out = _run_ffn_combine(buf, W1, W3, W2, wslot,
                           d0s.reshape(T, 1), d1s.reshape(T, 1))
    return out.reshape(bsz, seq, D), aux[0, 0]
